# baseline (speedup 1.0000x reference)
import functools

import jax
import jax.numpy as jnp
from jax import lax
from jax.experimental import pallas as pl
from jax.experimental.pallas import tpu as pltpu
from jax.experimental.pallas import tpu_sc as plsc

_NC = 2
_NS = 16
_NW = _NC * _NS

_D = 128
_K = 10


def _build_sc_gather(b_pad: int, chunk: int):
    b_per_w = b_pad // _NW
    assert b_per_w % chunk == 0
    n_chunks = b_per_w // chunk

    mesh = plsc.VectorSubcoreMesh(core_axis_name="c", subcore_axis_name="s")

    @functools.partial(
        pl.kernel,
        mesh=mesh,
        out_type=(
            jax.ShapeDtypeStruct((b_pad, _D), jnp.float32),
            jax.ShapeDtypeStruct((b_pad, _D), jnp.float32),
        ),
        scratch_types=[
            pltpu.VMEM((chunk,), jnp.int32),
            pltpu.VMEM((chunk * _K,), jnp.int32),
            pltpu.VMEM((chunk, _D), jnp.float32),
            pltpu.VMEM((chunk * _K, _D), jnp.float32),
            pltpu.VMEM((chunk, _D), jnp.float32),
            pltpu.SemaphoreType.DMA,
            pltpu.SemaphoreType.DMA,
            pltpu.SemaphoreType.DMA,
        ],
    )
    def sc_gather(nodes_hbm, neigh_hbm, table_hbm, self_out, nsum_out,
                  sidx_v, nidx_v, srows_v, nrows_v, nsum_v, isem, ssem, nsem):
        wid = lax.axis_index("s") * _NC + lax.axis_index("c")
        base = wid * b_per_w

        @pl.loop(0, n_chunks)
        def _chunk_loop(g):
            off = base + g * chunk

            pltpu.async_copy(nodes_hbm.at[pl.ds(off, chunk)], sidx_v, isem)
            pltpu.async_copy(neigh_hbm.at[pl.ds(off * _K, chunk * _K)],
                             nidx_v, isem)
            pltpu.make_async_copy(nodes_hbm.at[pl.ds(off, chunk)], sidx_v,
                                  isem).wait()
            pltpu.make_async_copy(neigh_hbm.at[pl.ds(0, chunk * _K)], nidx_v,
                                  isem).wait()

            pltpu.async_copy(table_hbm.at[sidx_v], srows_v, ssem)
            pltpu.async_copy(table_hbm.at[nidx_v], nrows_v, nsem)
            pltpu.make_async_copy(table_hbm.at[sidx_v], srows_v, ssem).wait()
            pltpu.sync_copy(srows_v, self_out.at[pl.ds(off, chunk)])
            pltpu.make_async_copy(table_hbm.at[nidx_v], nrows_v, nsem).wait()

            @pl.loop(0, chunk)
            def _node_loop(i):
                r0 = i * _K
                for c in range(_D // 16):
                    sl = pl.ds(c * 16, 16)
                    acc = nrows_v[r0, sl]
                    for j in range(1, _K):
                        acc = acc + nrows_v[r0 + j, sl]
                    nsum_v[i, sl] = acc

            pltpu.sync_copy(nsum_v, nsum_out.at[pl.ds(off, chunk)])

    return sc_gather


def _tc_matmul_body(s_ref, n_ref, w_ref, o_ref):
    s = s_ref[...]
    n = n_ref[...] * (1.0 / _K)
    acc = jnp.dot(s, w_ref[0:_D, :], preferred_element_type=jnp.float32)
    acc = acc + jnp.dot(n, w_ref[_D:2 * _D, :], preferred_element_type=jnp.float32)
    o_ref[...] = jnp.maximum(acc, 0.0)


def _tc_matmul(self_rows, nsum, w, bm: int):
    b_pad = self_rows.shape[0]
    grid = (b_pad // bm,)
    return pl.pallas_call(
        _tc_matmul_body,
        grid=grid,
        in_specs=[
            pl.BlockSpec((bm, _D), lambda i: (i, 0)),
            pl.BlockSpec((bm, _D), lambda i: (i, 0)),
            pl.BlockSpec((2 * _D, _D), lambda i: (0, 0)),
        ],
        out_specs=pl.BlockSpec((bm, _D), lambda i: (i, 0)),
        out_shape=jax.ShapeDtypeStruct((b_pad, _D), jnp.float32),
    )(self_rows, nsum, w)


def kernel(nodes, neigh_idx, feat_table, W):
    b = nodes.shape[0]

    chunk = 80
    unit = _NW * chunk
    b_pad = ((b + unit - 1) // unit) * unit
    pad = b_pad - b

    nodes_p = jnp.pad(nodes, (0, pad))
    neigh_flat = jnp.pad(neigh_idx.reshape(-1), (0, pad * _K))

    sc = _build_sc_gather(b_pad, chunk)
    self_rows, nsum = sc(nodes_p, neigh_flat, feat_table)

    out = _tc_matmul(self_rows, nsum, W, bm=1024)
    return out[:b]


# speedup vs baseline: 1.8768x; 1.8768x over previous
import functools

import jax
import jax.numpy as jnp
from jax import lax
from jax.experimental import pallas as pl
from jax.experimental.pallas import tpu as pltpu
from jax.experimental.pallas import tpu_sc as plsc

_NC = 2
_NS = 16
_NW = _NC * _NS

_D = 128
_K = 10


def _build_sc_gather(b_pad: int, chunk: int, m0: int, m1: int):
    assert b_pad == _NS * (m0 + m1) * chunk

    mesh = plsc.VectorSubcoreMesh(core_axis_name="c", subcore_axis_name="s")

    @functools.partial(
        pl.kernel,
        mesh=mesh,
        out_type=(
            jax.ShapeDtypeStruct((b_pad, _D), jnp.float32),
            jax.ShapeDtypeStruct((b_pad, _D), jnp.float32),
        ),
        scratch_types=[
            pltpu.VMEM((chunk,), jnp.int32),
            pltpu.VMEM((chunk * _K,), jnp.int32),
            pltpu.VMEM((chunk, _D), jnp.float32),
            pltpu.VMEM((chunk * _K, _D), jnp.float32),
            pltpu.VMEM((chunk, _D), jnp.float32),
            pltpu.SemaphoreType.DMA,
        ],
    )
    def sc_gather(nodes_hbm, neigh_hbm, table_hbm, self_out, nsum_out,
                  sidx_v, nidx_v, srows_v, nrows_v, nsum_v, sem):
        c = lax.axis_index("c")
        s = lax.axis_index("s")
        is0 = c == 0
        my_chunks = jnp.where(is0, m0, m1)
        base = jnp.where(is0, s * (m0 * chunk),
                         _NS * (m0 * chunk) + s * (m1 * chunk))

        @pl.loop(0, my_chunks)
        def _chunk_loop(g):
            off = base + g * chunk

            pltpu.sync_copy(nodes_hbm.at[pl.ds(off, chunk)], sidx_v)
            pltpu.async_copy(table_hbm.at[sidx_v], srows_v, sem).wait()
            pltpu.sync_copy(srows_v, self_out.at[pl.ds(off, chunk)])

            pltpu.sync_copy(neigh_hbm.at[pl.ds(off * _K, chunk * _K)], nidx_v)
            pltpu.async_copy(table_hbm.at[nidx_v], nrows_v, sem).wait()

            @pl.loop(0, chunk)
            def _node_loop(i):
                r0 = i * _K
                for cc in range(_D // 16):
                    sl = pl.ds(cc * 16, 16)
                    acc = nrows_v[r0, sl]
                    for j in range(1, _K):
                        acc = acc + nrows_v[r0 + j, sl]
                    nsum_v[i, sl] = acc

            pltpu.sync_copy(nsum_v, nsum_out.at[pl.ds(off, chunk)])

    return sc_gather


def _tc_matmul_body(s_ref, n_ref, w_ref, o_ref):
    s = s_ref[...]
    n = n_ref[...] * (1.0 / _K)
    acc = jnp.dot(s, w_ref[0:_D, :], preferred_element_type=jnp.float32)
    acc = acc + jnp.dot(n, w_ref[_D:2 * _D, :], preferred_element_type=jnp.float32)
    o_ref[...] = jnp.maximum(acc, 0.0)


def _tc_matmul(self_rows, nsum, w, bm: int):
    b_pad = self_rows.shape[0]
    grid = (b_pad // bm,)
    return pl.pallas_call(
        _tc_matmul_body,
        grid=grid,
        in_specs=[
            pl.BlockSpec((bm, _D), lambda i: (i, 0)),
            pl.BlockSpec((bm, _D), lambda i: (i, 0)),
            pl.BlockSpec((2 * _D, _D), lambda i: (0, 0)),
        ],
        out_specs=pl.BlockSpec((bm, _D), lambda i: (i, 0)),
        out_shape=jax.ShapeDtypeStruct((b_pad, _D), jnp.float32),
    )(self_rows, nsum, w)


def kernel(nodes, neigh_idx, feat_table, W):
    b = nodes.shape[0]

    chunk = 56
    m_total = -(-b // (_NS * chunk))
    m0, m1 = m_total // 2 + m_total % 2, m_total // 2
    b_pad = _NS * m_total * chunk
    pad = b_pad - b

    nodes_p = jnp.pad(nodes, (0, pad))
    neigh_flat = jnp.pad(neigh_idx.reshape(-1), (0, pad * _K))

    sc = _build_sc_gather(b_pad, chunk, m0, m1)
    self_rows, nsum = sc(nodes_p, neigh_flat, feat_table)

    out = _tc_matmul(self_rows, nsum, W, bm=1024)
    return out[:b]


# speedup vs baseline: 2.4073x; 1.2827x over previous
import functools

import jax
import jax.numpy as jnp
from jax import lax
from jax.experimental import pallas as pl
from jax.experimental.pallas import tpu as pltpu
from jax.experimental.pallas import tpu_sc as plsc

_NC = 2
_NS = 16

_D = 128
_K = 10

_C = 32


def _sum_k_rows(nr, nsum_v, chunk):
    @pl.loop(0, chunk)
    def _node_loop(i):
        r0 = i * _K
        for cc in range(_D // 16):
            sl = pl.ds(cc * 16, 16)
            acc = nr[r0, sl]
            for j in range(1, _K):
                acc = acc + nr[r0 + j, sl]
            nsum_v[i, sl] = acc


def _build_sc_gather(b_pad: int, m0: int, m1: int):
    assert b_pad == _NS * (m0 + m1) * _C
    assert m0 % 2 == 0 and m1 % 2 == 0
    core0_len = _NS * m0 * _C

    mesh = plsc.VectorSubcoreMesh(core_axis_name="c", subcore_axis_name="s")

    @functools.partial(
        pl.kernel,
        mesh=mesh,
        out_type=(
            jax.ShapeDtypeStruct((b_pad, _D), jnp.float32),
            jax.ShapeDtypeStruct((b_pad, _D), jnp.float32),
        ),
        scratch_types=[
            pltpu.VMEM((_C,), jnp.int32),
            pltpu.VMEM((_C,), jnp.int32),
            pltpu.VMEM((_C * _K,), jnp.int32),
            pltpu.VMEM((_C * _K,), jnp.int32),
            pltpu.VMEM((_C, _D), jnp.float32),
            pltpu.VMEM((_C, _D), jnp.float32),
            pltpu.VMEM((_C * _K, _D), jnp.float32),
            pltpu.VMEM((_C * _K, _D), jnp.float32),
            pltpu.VMEM((_C, _D), jnp.float32),
            pltpu.SemaphoreType.DMA,
            pltpu.SemaphoreType.DMA,
            pltpu.SemaphoreType.DMA,
            pltpu.SemaphoreType.DMA,
            pltpu.SemaphoreType.DMA,
            pltpu.SemaphoreType.DMA,
        ],
    )
    def sc_gather(nodes_hbm, neigh_hbm, table_hbm, self_out, nsum_out,
                  sidx0, sidx1, nidx0, nidx1, srows0, srows1, nrows0, nrows1,
                  nsum_v, isem0, isem1, ssem0, ssem1, nsem0, nsem1):
        c = lax.axis_index("c")
        s = lax.axis_index("s")

        is0 = c == 0
        mw = jnp.where(is0, m0, m1)
        base = jnp.where(is0, s * (m0 * _C), core0_len + s * (m1 * _C))
        sidx = (sidx0, sidx1)
        nidx = (nidx0, nidx1)
        srows = (srows0, srows1)
        nrows = (nrows0, nrows1)
        isem = (isem0, isem1)
        ssem = (ssem0, ssem1)
        nsem = (nsem0, nsem1)

        def idx_load(g, p):
            off = base + g * _C
            pltpu.async_copy(nodes_hbm.at[pl.ds(off, _C)], sidx[p], isem[p])
            pltpu.async_copy(neigh_hbm.at[pl.ds(off * _K, _C * _K)],
                             nidx[p], isem[p])

        def gather(g, p):
            pltpu.make_async_copy(nodes_hbm.at[pl.ds(0, _C)], sidx[p],
                                  isem[p]).wait()
            pltpu.make_async_copy(neigh_hbm.at[pl.ds(0, _C * _K)], nidx[p],
                                  isem[p]).wait()
            pltpu.async_copy(table_hbm.at[sidx[p]], srows[p], ssem[p])
            pltpu.async_copy(table_hbm.at[nidx[p]], nrows[p], nsem[p])

        def gather_wait(p):
            pltpu.make_async_copy(table_hbm.at[sidx[p]], srows[p],
                                  ssem[p]).wait()
            pltpu.make_async_copy(table_hbm.at[nidx[p]], nrows[p],
                                  nsem[p]).wait()

        def compute(g, p):
            off = base + g * _C
            pltpu.sync_copy(srows[p], self_out.at[pl.ds(off, _C)])
            _sum_k_rows(nrows[p], nsum_v, _C)
            pltpu.sync_copy(nsum_v, nsum_out.at[pl.ds(off, _C)])

        idx_load(0, 0)
        idx_load(1, 1)
        gather(0, 0)

        @pl.loop(0, mw - 2, step=2)
        def _chunk_loop(g):
            gather(g + 1, 1)
            gather_wait(0)
            idx_load(g + 2, 0)
            compute(g, 0)
            gather(g + 2, 0)
            gather_wait(1)
            idx_load(g + 3, 1)
            compute(g + 1, 1)

        gather(mw - 1, 1)
        gather_wait(0)
        compute(mw - 2, 0)
        gather_wait(1)
        compute(mw - 1, 1)

    return sc_gather


def _tc_matmul_body(s_ref, n_ref, w_ref, o_ref):
    s = s_ref[...]
    n = n_ref[...] * (1.0 / _K)
    acc = jnp.dot(s, w_ref[0:_D, :], preferred_element_type=jnp.float32)
    acc = acc + jnp.dot(n, w_ref[_D:2 * _D, :], preferred_element_type=jnp.float32)
    o_ref[...] = jnp.maximum(acc, 0.0)


def _tc_matmul(self_rows, nsum, w, bm: int):
    b_pad = self_rows.shape[0]
    grid = (b_pad // bm,)
    return pl.pallas_call(
        _tc_matmul_body,
        grid=grid,
        in_specs=[
            pl.BlockSpec((bm, _D), lambda i: (i, 0)),
            pl.BlockSpec((bm, _D), lambda i: (i, 0)),
            pl.BlockSpec((2 * _D, _D), lambda i: (0, 0)),
        ],
        out_specs=pl.BlockSpec((bm, _D), lambda i: (i, 0)),
        out_shape=jax.ShapeDtypeStruct((b_pad, _D), jnp.float32),
    )(self_rows, nsum, w)


def kernel(nodes, neigh_idx, feat_table, W):
    b = nodes.shape[0]

    share0 = 0.78
    m0 = max(2, 2 * round(share0 * b / (_NS * _C * 2)))
    rem = b - _NS * m0 * _C
    m1 = max(2, 2 * (-(-rem // (_NS * _C * 2))))
    b_pad = _NS * (m0 + m1) * _C
    pad = b_pad - b

    nodes_p = jnp.pad(nodes, (0, pad))
    neigh_flat = jnp.pad(neigh_idx.reshape(-1), (0, pad * _K))

    sc = _build_sc_gather(b_pad, m0, m1)
    self_rows, nsum = sc(nodes_p, neigh_flat, feat_table)

    bm = 128
    while b_pad % (2 * bm) == 0 and bm < 1024:
        bm *= 2
    out = _tc_matmul(self_rows, nsum, W, bm=bm)
    return out[:b]


# speedup vs baseline: 2.7124x; 1.1267x over previous
import functools

import jax
import jax.numpy as jnp
from jax import lax
from jax.experimental import pallas as pl
from jax.experimental.pallas import tpu as pltpu
from jax.experimental.pallas import tpu_sc as plsc

_NC = 2
_NS = 16

_D = 128
_K = 10

_C = 32


def _sum_k_rows(nr, nsum_v, chunk):
    @pl.loop(0, chunk)
    def _node_loop(i):
        r0 = i * _K
        for cc in range(_D // 16):
            sl = pl.ds(cc * 16, 16)
            acc = nr[r0, sl]
            for j in range(1, _K):
                acc = acc + nr[r0 + j, sl]
            nsum_v[i, sl] = acc


def _build_sc_gather(b: int, m0: int, m1: int):
    assert _NS * (m0 + m1) * _C >= b
    assert m0 % 2 == 0 and m1 % 2 == 0
    assert b % 16 == 0 and b >= _C
    core0_len = _NS * m0 * _C

    mesh = plsc.VectorSubcoreMesh(core_axis_name="c", subcore_axis_name="s")

    @functools.partial(
        pl.kernel,
        mesh=mesh,
        out_type=(
            jax.ShapeDtypeStruct((b, _D), jnp.float32),
            jax.ShapeDtypeStruct((b, _D), jnp.float32),
        ),
        scratch_types=[
            pltpu.VMEM((_C,), jnp.int32),
            pltpu.VMEM((_C,), jnp.int32),
            pltpu.VMEM((_C * _K,), jnp.int32),
            pltpu.VMEM((_C * _K,), jnp.int32),
            pltpu.VMEM((_C, _D), jnp.float32),
            pltpu.VMEM((_C, _D), jnp.float32),
            pltpu.VMEM((_C * _K, _D), jnp.float32),
            pltpu.VMEM((_C * _K, _D), jnp.float32),
            pltpu.VMEM((_C, _D), jnp.float32),
            pltpu.SemaphoreType.DMA,
            pltpu.SemaphoreType.DMA,
            pltpu.SemaphoreType.DMA,
            pltpu.SemaphoreType.DMA,
            pltpu.SemaphoreType.DMA,
            pltpu.SemaphoreType.DMA,
        ],
    )
    def sc_gather(nodes_hbm, neigh_hbm, table_hbm, self_out, nsum_out,
                  sidx0, sidx1, nidx0, nidx1, srows0, srows1, nrows0, nrows1,
                  nsum_v, isem0, isem1, ssem0, ssem1, nsem0, nsem1):
        c = lax.axis_index("c")
        s = lax.axis_index("s")

        is0 = c == 0
        mw = jnp.where(is0, m0, m1)
        base = jnp.where(is0, s * (m0 * _C), core0_len + s * (m1 * _C))
        sidx = (sidx0, sidx1)
        nidx = (nidx0, nidx1)
        srows = (srows0, srows1)
        nrows = (nrows0, nrows1)
        isem = (isem0, isem1)
        ssem = (ssem0, ssem1)
        nsem = (nsem0, nsem1)

        def chunk_off(g):
            off = jnp.minimum(base + g * _C, b - _C)
            return pl.multiple_of(off, 16)

        def idx_load(g, p):
            off = chunk_off(g)
            pltpu.async_copy(nodes_hbm.at[pl.ds(off, _C)], sidx[p], isem[p])
            pltpu.async_copy(neigh_hbm.at[pl.ds(off * _K, _C * _K)],
                             nidx[p], isem[p])

        def gather(g, p):
            pltpu.make_async_copy(nodes_hbm.at[pl.ds(0, _C)], sidx[p],
                                  isem[p]).wait()
            pltpu.make_async_copy(neigh_hbm.at[pl.ds(0, _C * _K)], nidx[p],
                                  isem[p]).wait()
            pltpu.async_copy(table_hbm.at[sidx[p]], srows[p], ssem[p])
            pltpu.async_copy(table_hbm.at[nidx[p]], nrows[p], nsem[p])

        def gather_wait(p):
            pltpu.make_async_copy(table_hbm.at[sidx[p]], srows[p],
                                  ssem[p]).wait()
            pltpu.make_async_copy(table_hbm.at[nidx[p]], nrows[p],
                                  nsem[p]).wait()

        def compute(g, p):
            off = chunk_off(g)
            pltpu.sync_copy(srows[p], self_out.at[pl.ds(off, _C)])
            _sum_k_rows(nrows[p], nsum_v, _C)
            pltpu.sync_copy(nsum_v, nsum_out.at[pl.ds(off, _C)])

        idx_load(0, 0)
        idx_load(1, 1)
        gather(0, 0)

        @pl.loop(0, mw - 2, step=2)
        def _chunk_loop(g):
            gather(g + 1, 1)
            gather_wait(0)
            idx_load(g + 2, 0)
            compute(g, 0)
            gather(g + 2, 0)
            gather_wait(1)
            idx_load(g + 3, 1)
            compute(g + 1, 1)

        gather(mw - 1, 1)
        gather_wait(0)
        compute(mw - 2, 0)
        gather_wait(1)
        compute(mw - 1, 1)

    return sc_gather


def _tc_matmul_body(s_ref, n_ref, w_ref, o_ref):
    s = s_ref[...]
    n = n_ref[...] * (1.0 / _K)
    acc = jnp.dot(s, w_ref[0:_D, :], preferred_element_type=jnp.float32)
    acc = acc + jnp.dot(n, w_ref[_D:2 * _D, :], preferred_element_type=jnp.float32)
    o_ref[...] = jnp.maximum(acc, 0.0)


def _tc_matmul(self_rows, nsum, w, bm: int):
    b_pad = self_rows.shape[0]
    grid = (b_pad // bm,)
    return pl.pallas_call(
        _tc_matmul_body,
        grid=grid,
        in_specs=[
            pl.BlockSpec((bm, _D), lambda i: (i, 0)),
            pl.BlockSpec((bm, _D), lambda i: (i, 0)),
            pl.BlockSpec((2 * _D, _D), lambda i: (0, 0)),
        ],
        out_specs=pl.BlockSpec((bm, _D), lambda i: (i, 0)),
        out_shape=jax.ShapeDtypeStruct((b_pad, _D), jnp.float32),
    )(self_rows, nsum, w)


def kernel(nodes, neigh_idx, feat_table, W):
    b = nodes.shape[0]

    share0 = 0.69
    m0 = max(2, 2 * round(share0 * b / (_NS * _C * 2)))
    rem = b - _NS * m0 * _C
    m1 = max(2, 2 * (-(-rem // (_NS * _C * 2))))

    neigh_flat = neigh_idx.reshape(-1)

    sc = _build_sc_gather(b, m0, m1)
    self_rows, nsum = sc(nodes, neigh_flat, feat_table)

    bm = 8
    for cand in (1024, 512, 1000, 400, 256, 200, 128, 80, 16):
        if b % cand == 0:
            bm = cand
            break
    out = _tc_matmul(self_rows, nsum, W, bm=bm)
    return out


# speedup vs baseline: 3.0269x; 1.1160x over previous
import functools

import jax
import jax.numpy as jnp
from jax import lax
from jax.experimental import pallas as pl
from jax.experimental.pallas import tpu as pltpu
from jax.experimental.pallas import tpu_sc as plsc

_NC = 2
_NS = 16

_D = 128
_K = 10

_C = 32


def _sum_k_rows(nr, nsum_v, chunk):
    @pl.loop(0, chunk)
    def _node_loop(i):
        r0 = i * _K
        for cc in range(_D // 16):
            sl = pl.ds(cc * 16, 16)
            acc = nr[r0, sl]
            for j in range(1, _K):
                acc = acc + nr[r0 + j, sl]
            nsum_v[i, sl] = acc


def _build_sc_gather(b: int, m0: int, m1: int):
    assert _NS * (m0 + m1) * _C >= b
    assert m0 % 2 == 0 and m1 % 2 == 0
    assert b % 16 == 0 and b >= _C
    core0_len = _NS * m0 * _C

    mesh = plsc.VectorSubcoreMesh(core_axis_name="c", subcore_axis_name="s")

    @functools.partial(
        pl.kernel,
        mesh=mesh,
        compiler_params=pltpu.CompilerParams(use_tc_tiling_on_sc=True),
        out_type=(
            jax.ShapeDtypeStruct((b, _D), jnp.float32),
            jax.ShapeDtypeStruct((b, _D), jnp.float32),
        ),
        scratch_types=[
            pltpu.VMEM((_C,), jnp.int32),
            pltpu.VMEM((_C,), jnp.int32),
            pltpu.VMEM((_C * _K,), jnp.int32),
            pltpu.VMEM((_C * _K,), jnp.int32),
            pltpu.VMEM((_C, _D), jnp.float32),
            pltpu.VMEM((_C, _D), jnp.float32),
            pltpu.VMEM((_C * _K, _D), jnp.float32),
            pltpu.VMEM((_C * _K, _D), jnp.float32),
            pltpu.VMEM((_C, _D), jnp.float32),
            pltpu.SemaphoreType.DMA,
            pltpu.SemaphoreType.DMA,
            pltpu.SemaphoreType.DMA,
            pltpu.SemaphoreType.DMA,
            pltpu.SemaphoreType.DMA,
            pltpu.SemaphoreType.DMA,
        ],
    )
    def sc_gather(nodes_hbm, neigh_hbm, table_hbm, self_out, nsum_out,
                  sidx0, sidx1, nidx0, nidx1, srows0, srows1, nrows0, nrows1,
                  nsum_v, isem0, isem1, ssem0, ssem1, nsem0, nsem1):
        c = lax.axis_index("c")
        s = lax.axis_index("s")

        is0 = c == 0
        mw = jnp.where(is0, m0, m1)
        base = jnp.where(is0, s * (m0 * _C), core0_len + s * (m1 * _C))
        sidx = (sidx0, sidx1)
        nidx = (nidx0, nidx1)
        srows = (srows0, srows1)
        nrows = (nrows0, nrows1)
        isem = (isem0, isem1)
        ssem = (ssem0, ssem1)
        nsem = (nsem0, nsem1)

        def chunk_off(g):
            off = jnp.minimum(base + g * _C, b - _C)
            return pl.multiple_of(off, 16)

        def idx_load(g, p):
            off = chunk_off(g)
            pltpu.async_copy(nodes_hbm.at[pl.ds(off, _C)], sidx[p], isem[p])
            pltpu.async_copy(neigh_hbm.at[pl.ds(off * _K, _C * _K)],
                             nidx[p], isem[p])

        def gather(g, p):
            pltpu.make_async_copy(nodes_hbm.at[pl.ds(0, _C)], sidx[p],
                                  isem[p]).wait()
            pltpu.make_async_copy(neigh_hbm.at[pl.ds(0, _C * _K)], nidx[p],
                                  isem[p]).wait()
            pltpu.async_copy(table_hbm.at[sidx[p]], srows[p], ssem[p])
            pltpu.async_copy(table_hbm.at[nidx[p]], nrows[p], nsem[p])

        def gather_wait(p):
            pltpu.make_async_copy(table_hbm.at[sidx[p]], srows[p],
                                  ssem[p]).wait()
            pltpu.make_async_copy(table_hbm.at[nidx[p]], nrows[p],
                                  nsem[p]).wait()

        def compute(g, p):
            off = chunk_off(g)
            pltpu.sync_copy(srows[p], self_out.at[pl.ds(off, _C)])
            _sum_k_rows(nrows[p], nsum_v, _C)
            pltpu.sync_copy(nsum_v, nsum_out.at[pl.ds(off, _C)])

        idx_load(0, 0)
        idx_load(1, 1)
        gather(0, 0)

        @pl.loop(0, mw - 2, step=2)
        def _chunk_loop(g):
            gather(g + 1, 1)
            gather_wait(0)
            idx_load(g + 2, 0)
            compute(g, 0)
            gather(g + 2, 0)
            gather_wait(1)
            idx_load(g + 3, 1)
            compute(g + 1, 1)

        gather(mw - 1, 1)
        gather_wait(0)
        compute(mw - 2, 0)
        gather_wait(1)
        compute(mw - 1, 1)

    return sc_gather


def _tc_matmul_body(s_ref, n_ref, w_ref, o_ref):
    s = s_ref[...]
    n = n_ref[...] * (1.0 / _K)
    acc = jnp.dot(s, w_ref[0:_D, :], preferred_element_type=jnp.float32)
    acc = acc + jnp.dot(n, w_ref[_D:2 * _D, :], preferred_element_type=jnp.float32)
    o_ref[...] = jnp.maximum(acc, 0.0)


def _tc_matmul(self_rows, nsum, w, bm: int):
    b_pad = self_rows.shape[0]
    grid = (b_pad // bm,)
    return pl.pallas_call(
        _tc_matmul_body,
        grid=grid,
        in_specs=[
            pl.BlockSpec((bm, _D), lambda i: (i, 0)),
            pl.BlockSpec((bm, _D), lambda i: (i, 0)),
            pl.BlockSpec((2 * _D, _D), lambda i: (0, 0)),
        ],
        out_specs=pl.BlockSpec((bm, _D), lambda i: (i, 0)),
        out_shape=jax.ShapeDtypeStruct((b_pad, _D), jnp.float32),
    )(self_rows, nsum, w)


def kernel(nodes, neigh_idx, feat_table, W):
    b = nodes.shape[0]

    share0 = 0.60
    m0 = max(2, 2 * round(share0 * b / (_NS * _C * 2)))
    rem = b - _NS * m0 * _C
    m1 = max(2, 2 * (-(-rem // (_NS * _C * 2))))

    neigh_flat = neigh_idx.reshape(-1)

    sc = _build_sc_gather(b, m0, m1)
    self_rows, nsum = sc(nodes, neigh_flat, feat_table)

    bm = 8
    for cand in (1024, 512, 1000, 400, 256, 200, 128, 80, 16):
        if b % cand == 0:
            bm = cand
            break
    out = _tc_matmul(self_rows, nsum, W, bm=bm)
    return out


# speedup vs baseline: 3.4208x; 1.1301x over previous
import functools

import jax
import jax.numpy as jnp
from jax import lax
from jax.experimental import pallas as pl
from jax.experimental.pallas import tpu as pltpu
from jax.experimental.pallas import tpu_sc as plsc

_NC = 2
_NS = 16

_D = 128
_K = 10

_C = 32


def _sum_k_rows(nr, nsum_v, chunk):
    @pl.loop(0, chunk)
    def _node_loop(i):
        r0 = i * _K
        for cc in range(_D // 16):
            sl = pl.ds(cc * 16, 16)
            acc = nr[r0, sl]
            for j in range(1, _K):
                acc = acc + nr[r0 + j, sl]
            nsum_v[i, sl] = acc


def _build_sc_gather(b: int, m0: int, m1: int):
    assert _NS * (m0 + m1) * _C >= b
    assert m0 % 2 == 0 and m1 % 2 == 0
    assert b % 16 == 0 and b >= _C
    core0_len = _NS * m0 * _C

    mesh = plsc.VectorSubcoreMesh(core_axis_name="c", subcore_axis_name="s")

    @functools.partial(
        pl.kernel,
        mesh=mesh,
        compiler_params=pltpu.CompilerParams(use_tc_tiling_on_sc=True),
        out_type=(
            jax.ShapeDtypeStruct((b, _D), jnp.float32),
            jax.ShapeDtypeStruct((b, _D), jnp.float32),
        ),
        scratch_types=[
            pltpu.VMEM((_C,), jnp.int32),
            pltpu.VMEM((_C,), jnp.int32),
            pltpu.VMEM((_C * _K,), jnp.int32),
            pltpu.VMEM((_C * _K,), jnp.int32),
            pltpu.VMEM((_C, _D), jnp.float32),
            pltpu.VMEM((_C, _D), jnp.float32),
            pltpu.VMEM((_C * _K, _D), jnp.float32),
            pltpu.VMEM((_C * _K, _D), jnp.float32),
            pltpu.VMEM((_C, _D), jnp.float32),
            pltpu.SemaphoreType.DMA,
            pltpu.SemaphoreType.DMA,
            pltpu.SemaphoreType.DMA,
            pltpu.SemaphoreType.DMA,
            pltpu.SemaphoreType.DMA,
            pltpu.SemaphoreType.DMA,
        ],
    )
    def sc_gather(nodes_hbm, neigh_hbm, table_hbm, self_out, nsum_out,
                  sidx0, sidx1, nidx0, nidx1, srows0, srows1, nrows0, nrows1,
                  nsum_v, isem0, isem1, ssem0, ssem1, nsem0, nsem1):
        c = lax.axis_index("c")
        s = lax.axis_index("s")

        is0 = c == 0
        mw = jnp.where(is0, m0, m1)
        base = jnp.where(is0, s * (m0 * _C), core0_len + s * (m1 * _C))
        sidx = (sidx0, sidx1)
        nidx = (nidx0, nidx1)
        srows = (srows0, srows1)
        nrows = (nrows0, nrows1)
        isem = (isem0, isem1)
        ssem = (ssem0, ssem1)
        nsem = (nsem0, nsem1)

        def chunk_off(g):
            off = jnp.minimum(base + g * _C, b - _C)
            return pl.multiple_of(off, 16)

        def idx_load(g, p):
            off = chunk_off(g)
            pltpu.async_copy(nodes_hbm.at[pl.ds(off, _C)], sidx[p], isem[p])
            pltpu.async_copy(neigh_hbm.at[pl.ds(off * _K, _C * _K)],
                             nidx[p], isem[p])

        def gather(g, p):
            pltpu.make_async_copy(nodes_hbm.at[pl.ds(0, _C)], sidx[p],
                                  isem[p]).wait()
            pltpu.make_async_copy(neigh_hbm.at[pl.ds(0, _C * _K)], nidx[p],
                                  isem[p]).wait()
            pltpu.async_copy(table_hbm.at[sidx[p]], srows[p], ssem[p])
            pltpu.async_copy(table_hbm.at[nidx[p]], nrows[p], nsem[p])

        def gather_wait(p):
            pltpu.make_async_copy(table_hbm.at[sidx[p]], srows[p],
                                  ssem[p]).wait()
            pltpu.make_async_copy(table_hbm.at[nidx[p]], nrows[p],
                                  nsem[p]).wait()

        def compute(g, p):
            off = chunk_off(g)
            pltpu.sync_copy(srows[p], self_out.at[pl.ds(off, _C)])
            _sum_k_rows(nrows[p], nsum_v, _C)
            pltpu.sync_copy(nsum_v, nsum_out.at[pl.ds(off, _C)])

        idx_load(0, 0)
        idx_load(1, 1)
        gather(0, 0)

        @pl.loop(0, mw - 2, step=2)
        def _chunk_loop(g):
            gather(g + 1, 1)
            gather_wait(0)
            idx_load(g + 2, 0)
            compute(g, 0)
            gather(g + 2, 0)
            gather_wait(1)
            idx_load(g + 3, 1)
            compute(g + 1, 1)

        gather(mw - 1, 1)
        gather_wait(0)
        compute(mw - 2, 0)
        gather_wait(1)
        compute(mw - 1, 1)

    return sc_gather


def _tc_matmul_body(s_ref, n_ref, w_ref, o_ref):
    s = s_ref[...]
    n = n_ref[...] * (1.0 / _K)
    acc = jnp.dot(s, w_ref[0:_D, :], preferred_element_type=jnp.float32)
    acc = acc + jnp.dot(n, w_ref[_D:2 * _D, :], preferred_element_type=jnp.float32)
    o_ref[...] = jnp.maximum(acc, 0.0)


def _tc_matmul(self_rows, nsum, w, bm: int):
    b_pad = self_rows.shape[0]
    grid = (b_pad // bm,)
    return pl.pallas_call(
        _tc_matmul_body,
        grid=grid,
        in_specs=[
            pl.BlockSpec((bm, _D), lambda i: (i, 0)),
            pl.BlockSpec((bm, _D), lambda i: (i, 0)),
            pl.BlockSpec((2 * _D, _D), lambda i: (0, 0)),
        ],
        out_specs=pl.BlockSpec((bm, _D), lambda i: (i, 0)),
        out_shape=jax.ShapeDtypeStruct((b_pad, _D), jnp.float32),
    )(self_rows, nsum, w)


def kernel(nodes, neigh_idx, feat_table, W):
    b = nodes.shape[0]

    share0 = 0.54
    m0 = max(2, 2 * round(share0 * b / (_NS * _C * 2)))
    rem = b - _NS * m0 * _C
    m1 = max(2, 2 * (-(-rem // (_NS * _C * 2))))

    neigh_flat = neigh_idx.reshape(-1)

    sc = _build_sc_gather(b, m0, m1)
    self_rows, nsum = sc(nodes, neigh_flat, feat_table)

    bm = 8
    for cand in (2000, 1024, 512, 1000, 400, 256, 200, 128, 80, 16):
        if b % cand == 0:
            bm = cand
            break
    out = _tc_matmul(self_rows, nsum, W, bm=bm)
    return out
